# Initial kernel scaffold; baseline (speedup 1.0000x reference)
import jax
import jax.numpy as jnp
from jax.experimental import pallas as pl


def kernel(x, edge_index, W1, b1, W2, b2, Wfc, bfc):
    raise NotImplementedError("write your pallas kernel here")


# speedup vs baseline: 25.9104x; 25.9104x over previous
import functools

import jax
import jax.numpy as jnp
from jax import lax
from jax.experimental import pallas as pl
from jax.experimental.pallas import tpu as pltpu
from jax.experimental.pallas import tpu_sc as plsc

NN = 10000
EE = 320000
DD = 128
HH = 64
NC, NS, LL = 2, 16, 16
NWORK = NC * NS
CHUNK = 128
NCH = 79
EPAD = NWORK * NCH * CHUNK
ROWS_PER_TILE = 632
NP = NS * ROWS_PER_TILE
EW = EE // NWORK

_MESH = plsc.VectorSubcoreMesh(
    core_axis_name="c", subcore_axis_name="s", num_cores=NC, num_subcores=NS)


def _wid():
  return lax.axis_index("c") * NS + lax.axis_index("s")


@functools.partial(
    pl.kernel,
    out_type=jax.ShapeDtypeStruct((NC, NP, LL), jnp.float32),
    mesh=_MESH,
    compiler_params=pltpu.CompilerParams(use_tc_tiling_on_sc=False),
    scratch_types=[
        pltpu.VMEM((NCH, CHUNK), jnp.int32),
        pltpu.VMEM((CHUNK, LL), jnp.float32),
        pltpu.VMEM_SHARED((NP, LL), jnp.float32),
    ],
)
def _sc_degree(dst3, zeros16, ones16, cnt_out, dst_l, ones_v, acc):
  c = lax.axis_index("c")
  s = lax.axis_index("s")
  w = _wid()
  rbase = s * ROWS_PER_TILE
  pltpu.sync_copy(zeros16.at[pl.ds(rbase, ROWS_PER_TILE)],
                  acc.at[pl.ds(rbase, ROWS_PER_TILE)])
  pltpu.sync_copy(ones16, ones_v)
  pltpu.sync_copy(dst3.at[w], dst_l)
  plsc.subcore_barrier()

  @pl.loop(0, NCH)
  def _(j):
    pltpu.sync_copy(ones_v, acc.at[dst_l.at[j]], add=True)

  plsc.subcore_barrier()
  pltpu.sync_copy(acc.at[pl.ds(rbase, ROWS_PER_TILE)],
                  cnt_out.at[c, pl.ds(rbase, ROWS_PER_TILE)])


@functools.partial(
    pl.kernel,
    out_type=jax.ShapeDtypeStruct((NC, NP, HH), jnp.float32),
    mesh=_MESH,
    compiler_params=pltpu.CompilerParams(use_tc_tiling_on_sc=False),
    scratch_types=[
        pltpu.VMEM((NCH, CHUNK), jnp.int32),
        pltpu.VMEM((NCH, CHUNK), jnp.int32),
        pltpu.VMEM((CHUNK, HH), jnp.float32),
        pltpu.VMEM((CHUNK, HH), jnp.float32),
        pltpu.SemaphoreType.DMA,
        pltpu.SemaphoreType.DMA,
        pltpu.VMEM_SHARED((NP, HH), jnp.float32),
    ],
)
def _sc_msgpass(g_tab, src3, dst3, zeros64, s_out,
                src_l, dst_l, rows0, rows1, sem0, sem1, acc):
  c = lax.axis_index("c")
  s = lax.axis_index("s")
  w = _wid()
  rbase = s * ROWS_PER_TILE
  pltpu.sync_copy(zeros64.at[pl.ds(rbase, ROWS_PER_TILE)],
                  acc.at[pl.ds(rbase, ROWS_PER_TILE)])
  pltpu.sync_copy(src3.at[w], src_l)
  pltpu.sync_copy(dst3.at[w], dst_l)
  plsc.subcore_barrier()

  pltpu.async_copy(g_tab.at[src_l.at[0]], rows0, sem0)

  @pl.loop(0, NCH - 1)
  def _(j):
    even = j % 2 == 0

    def do(cur, nxt, sem_cur, sem_nxt):
      pltpu.async_copy(g_tab.at[src_l.at[j + 1]], nxt, sem_nxt)
      pltpu.make_async_copy(g_tab.at[src_l.at[j]], cur, sem_cur).wait()
      pltpu.sync_copy(cur, acc.at[dst_l.at[j]], add=True)

    @pl.when(even)
    def _():
      do(rows0, rows1, sem0, sem1)

    @pl.when(jnp.logical_not(even))
    def _():
      do(rows1, rows0, sem1, sem0)

  last = NCH - 1
  pltpu.make_async_copy(g_tab.at[src_l.at[last]], rows0, sem0).wait()
  pltpu.sync_copy(rows0, acc.at[dst_l.at[last]], add=True)

  plsc.subcore_barrier()
  pltpu.sync_copy(acc.at[pl.ds(rbase, ROWS_PER_TILE)],
                  s_out.at[c, pl.ds(rbase, ROWS_PER_TILE)])


@functools.partial(
    pl.kernel,
    out_type=jax.ShapeDtypeStruct((NWORK, EW), jnp.float32),
    mesh=_MESH,
    compiler_params=pltpu.CompilerParams(
        use_tc_tiling_on_sc=False, needs_layout_passes=False),
    scratch_types=[
        pltpu.VMEM((NN,), jnp.float32),
        pltpu.VMEM((NN,), jnp.float32),
        pltpu.VMEM((EW,), jnp.int32),
        pltpu.VMEM((EW,), jnp.int32),
        pltpu.VMEM((EW,), jnp.float32),
    ],
)
def _sc_decode(s_tab, t_tab, ei3, dec_out, s_l, t_l, src_l, dst_l, ob):
  w = _wid()
  pltpu.sync_copy(s_tab, s_l)
  pltpu.sync_copy(t_tab, t_l)
  pltpu.sync_copy(ei3.at[0, w], src_l)
  pltpu.sync_copy(ei3.at[1, w], dst_l)

  @pl.loop(0, EW // LL)
  def _(i):
    si = src_l[pl.ds(i * LL, LL)]
    di = dst_l[pl.ds(i * LL, LL)]
    sv = plsc.load_gather(s_l, [si])
    tv = plsc.load_gather(t_l, [di])
    y = sv + tv
    ob[pl.ds(i * LL, LL)] = 1.0 / (1.0 + jnp.exp(-y))

  pltpu.sync_copy(ob, dec_out.at[w])


def _dinv_from_cnt(cnt_ref):
  cnt = cnt_ref[0, 0:NN, 0:1] + cnt_ref[1, 0:NN, 0:1]
  return lax.rsqrt(cnt + 1.0)


def _tc_prep1_body(x_ref, w1_ref, b1_ref, cnt_ref, g1_ref, u1_ref):
  dinv = _dinv_from_cnt(cnt_ref)
  h = jnp.dot(x_ref[...], w1_ref[...], preferred_element_type=jnp.float32)
  g1_ref[...] = dinv * h
  u1_ref[...] = dinv * dinv * h + b1_ref[...]


def _tc_mid_body(sp_ref, u1_ref, w2_ref, b2_ref, cnt_ref, g2_ref, u2_ref):
  dinv = _dinv_from_cnt(cnt_ref)
  ssum = sp_ref[0, 0:NN, :] + sp_ref[1, 0:NN, :]
  z1 = jnp.maximum(dinv * ssum + u1_ref[...], 0.0)
  h2 = jnp.dot(z1, w2_ref[...], preferred_element_type=jnp.float32)
  g2_ref[...] = dinv * h2
  u2_ref[...] = dinv * dinv * h2 + b2_ref[...]


def _tc_fin_body(sp_ref, u2_ref, wfc_ref, bfc_ref, cnt_ref, s_ref, t_ref):
  dinv = _dinv_from_cnt(cnt_ref)
  ssum = sp_ref[0, 0:NN, :] + sp_ref[1, 0:NN, :]
  z2 = dinv * ssum + u2_ref[...]
  s_ref[...] = jnp.dot(z2, wfc_ref[0:HH, 0], preferred_element_type=jnp.float32) + bfc_ref[...]
  t_ref[...] = jnp.dot(z2, wfc_ref[HH:2 * HH, 0], preferred_element_type=jnp.float32)


_tc_prep1 = pl.pallas_call(
    _tc_prep1_body,
    out_shape=[jax.ShapeDtypeStruct((NN, HH), jnp.float32),
               jax.ShapeDtypeStruct((NN, HH), jnp.float32)],
)

_tc_mid = pl.pallas_call(
    _tc_mid_body,
    out_shape=[jax.ShapeDtypeStruct((NN, HH), jnp.float32),
               jax.ShapeDtypeStruct((NN, HH), jnp.float32)],
)

_tc_fin = pl.pallas_call(
    _tc_fin_body,
    out_shape=[jax.ShapeDtypeStruct((NN,), jnp.float32),
               jax.ShapeDtypeStruct((NN,), jnp.float32)],
)


def kernel(x, edge_index, W1, b1, W2, b2, Wfc, bfc):
  src = edge_index[0]
  dst = edge_index[1]
  npad = EPAD - EE
  srcp = jnp.concatenate([src, jnp.zeros((npad,), jnp.int32)])
  dstp = jnp.concatenate([dst, jnp.full((npad,), NN, jnp.int32)])
  src3 = srcp.reshape(NWORK, NCH, CHUNK)
  dst3 = dstp.reshape(NWORK, NCH, CHUNK)
  ei3 = edge_index.reshape(2, NWORK, EW)

  zeros16 = jnp.zeros((NP, LL), jnp.float32)
  ones16 = jnp.ones((CHUNK, LL), jnp.float32)
  zeros64 = jnp.zeros((NP, HH), jnp.float32)

  cnt_part = _sc_degree(dst3, zeros16, ones16)
  g1, u1 = _tc_prep1(x, W1, b1, cnt_part)
  s1_part = _sc_msgpass(g1, src3, dst3, zeros64)
  g2, u2 = _tc_mid(s1_part, u1, W2, b2, cnt_part)
  s2_part = _sc_msgpass(g2, src3, dst3, zeros64)
  s_tab, t_tab = _tc_fin(s2_part, u2, Wfc, bfc, cnt_part)
  dec = _sc_decode(s_tab, t_tab, ei3)
  return dec.reshape(EE, 1)


# speedup vs baseline: 35.0283x; 1.3519x over previous
import functools

import jax
import jax.numpy as jnp
from jax import lax
from jax.experimental import pallas as pl
from jax.experimental.pallas import tpu as pltpu
from jax.experimental.pallas import tpu_sc as plsc

NN = 10000
EE = 320000
DD = 128
HH = 64
NC, NS, LL = 2, 16, 16
NWORK = NC * NS
CHUNK = 128
NCH = 79
EPAD = NWORK * NCH * CHUNK
ROWS_PER_TILE = 632
NP = NS * ROWS_PER_TILE
EW = EE // NWORK

_MESH = plsc.VectorSubcoreMesh(
    core_axis_name="c", subcore_axis_name="s", num_cores=NC, num_subcores=NS)


def _wid():
  return lax.axis_index("c") * NS + lax.axis_index("s")


@functools.partial(
    pl.kernel,
    out_type=jax.ShapeDtypeStruct((NC, NP, LL), jnp.float32),
    mesh=_MESH,
    compiler_params=pltpu.CompilerParams(use_tc_tiling_on_sc=False),
    scratch_types=[
        pltpu.VMEM((NCH, CHUNK), jnp.int32),
        pltpu.VMEM((CHUNK, LL), jnp.float32),
        pltpu.VMEM_SHARED((NP, LL), jnp.float32),
    ],
)
def _sc_degree(dst3, zeros16, ones16, cnt_out, dst_l, ones_v, acc):
  c = lax.axis_index("c")
  s = lax.axis_index("s")
  w = _wid()
  rbase = s * ROWS_PER_TILE
  pltpu.sync_copy(zeros16.at[pl.ds(rbase, ROWS_PER_TILE)],
                  acc.at[pl.ds(rbase, ROWS_PER_TILE)])
  pltpu.sync_copy(ones16, ones_v)
  pltpu.sync_copy(dst3.at[w], dst_l)
  plsc.subcore_barrier()

  @pl.loop(0, NCH)
  def _(j):
    pltpu.sync_copy(ones_v, acc.at[dst_l.at[j]], add=True)

  plsc.subcore_barrier()
  pltpu.sync_copy(acc.at[pl.ds(rbase, ROWS_PER_TILE)],
                  cnt_out.at[c, pl.ds(rbase, ROWS_PER_TILE)])


@functools.partial(
    pl.kernel,
    out_type=jax.ShapeDtypeStruct((NC, NP, HH), jnp.float32),
    mesh=_MESH,
    compiler_params=pltpu.CompilerParams(use_tc_tiling_on_sc=False),
    scratch_types=[
        pltpu.VMEM((NCH, CHUNK), jnp.int32),
        pltpu.VMEM((NCH, CHUNK), jnp.int32),
        pltpu.VMEM((CHUNK, HH), jnp.float32),
        pltpu.VMEM((CHUNK, HH), jnp.float32),
        pltpu.SemaphoreType.DMA,
        pltpu.SemaphoreType.DMA,
        pltpu.VMEM_SHARED((NP, HH), jnp.float32),
        pltpu.VMEM_SHARED((NP, HH), jnp.float32),
    ],
)
def _sc_msgpass(g_tab, src3, dst3, zeros64, s_out,
                src_l, dst_l, rows0, rows1, sem0, sem1, acc, g_sp):
  c = lax.axis_index("c")
  s = lax.axis_index("s")
  w = _wid()
  rbase = s * ROWS_PER_TILE
  pltpu.sync_copy(g_tab.at[pl.ds(rbase, ROWS_PER_TILE)],
                  g_sp.at[pl.ds(rbase, ROWS_PER_TILE)])
  pltpu.sync_copy(zeros64.at[pl.ds(rbase, ROWS_PER_TILE)],
                  acc.at[pl.ds(rbase, ROWS_PER_TILE)])
  pltpu.sync_copy(src3.at[w], src_l)
  pltpu.sync_copy(dst3.at[w], dst_l)
  plsc.subcore_barrier()

  pltpu.async_copy(g_sp.at[src_l.at[0]], rows0, sem0)

  @pl.loop(0, NCH - 1)
  def _(j):
    even = j % 2 == 0

    def do(cur, nxt, sem_cur, sem_nxt):
      pltpu.async_copy(g_sp.at[src_l.at[j + 1]], nxt, sem_nxt)
      pltpu.make_async_copy(g_sp.at[src_l.at[j]], cur, sem_cur).wait()
      pltpu.sync_copy(cur, acc.at[dst_l.at[j]], add=True)

    @pl.when(even)
    def _():
      do(rows0, rows1, sem0, sem1)

    @pl.when(jnp.logical_not(even))
    def _():
      do(rows1, rows0, sem1, sem0)

  last = NCH - 1
  pltpu.make_async_copy(g_sp.at[src_l.at[last]], rows0, sem0).wait()
  pltpu.sync_copy(rows0, acc.at[dst_l.at[last]], add=True)

  plsc.subcore_barrier()
  pltpu.sync_copy(acc.at[pl.ds(rbase, ROWS_PER_TILE)],
                  s_out.at[c, pl.ds(rbase, ROWS_PER_TILE)])


@functools.partial(
    pl.kernel,
    out_type=jax.ShapeDtypeStruct((NWORK, EW), jnp.float32),
    mesh=_MESH,
    compiler_params=pltpu.CompilerParams(
        use_tc_tiling_on_sc=False, needs_layout_passes=False),
    scratch_types=[
        pltpu.VMEM((NN,), jnp.float32),
        pltpu.VMEM((NN,), jnp.float32),
        pltpu.VMEM((EW,), jnp.int32),
        pltpu.VMEM((EW,), jnp.int32),
        pltpu.VMEM((EW,), jnp.float32),
    ],
)
def _sc_decode(s_tab, t_tab, ei3, dec_out, s_l, t_l, src_l, dst_l, ob):
  w = _wid()
  pltpu.sync_copy(s_tab, s_l)
  pltpu.sync_copy(t_tab, t_l)
  pltpu.sync_copy(ei3.at[0, w], src_l)
  pltpu.sync_copy(ei3.at[1, w], dst_l)

  @pl.loop(0, EW // LL)
  def _(i):
    si = src_l[pl.ds(i * LL, LL)]
    di = dst_l[pl.ds(i * LL, LL)]
    sv = plsc.load_gather(s_l, [si])
    tv = plsc.load_gather(t_l, [di])
    y = sv + tv
    ob[pl.ds(i * LL, LL)] = 1.0 / (1.0 + jnp.exp(-y))

  pltpu.sync_copy(ob, dec_out.at[w])


def _dinv_from_cnt(cnt_ref):
  cnt = cnt_ref[0, 0:NN, 0:1] + cnt_ref[1, 0:NN, 0:1]
  return lax.rsqrt(cnt + 1.0)


def _tc_prep1_body(x_ref, w1_ref, b1_ref, cnt_ref, g1_ref, u1_ref):
  dinv = _dinv_from_cnt(cnt_ref)
  h = jnp.dot(x_ref[...], w1_ref[...], preferred_element_type=jnp.float32)
  g1_ref[...] = jnp.concatenate(
      [dinv * h, jnp.zeros((NP - NN, HH), jnp.float32)], axis=0)
  u1_ref[...] = dinv * dinv * h + b1_ref[...]


def _tc_mid_body(sp_ref, u1_ref, w2_ref, b2_ref, cnt_ref, g2_ref, u2_ref):
  dinv = _dinv_from_cnt(cnt_ref)
  ssum = sp_ref[0, 0:NN, :] + sp_ref[1, 0:NN, :]
  z1 = jnp.maximum(dinv * ssum + u1_ref[...], 0.0)
  h2 = jnp.dot(z1, w2_ref[...], preferred_element_type=jnp.float32)
  g2_ref[...] = jnp.concatenate(
      [dinv * h2, jnp.zeros((NP - NN, HH), jnp.float32)], axis=0)
  u2_ref[...] = dinv * dinv * h2 + b2_ref[...]


def _tc_fin_body(sp_ref, u2_ref, wfc_ref, bfc_ref, cnt_ref, s_ref, t_ref):
  dinv = _dinv_from_cnt(cnt_ref)
  ssum = sp_ref[0, 0:NN, :] + sp_ref[1, 0:NN, :]
  z2 = dinv * ssum + u2_ref[...]
  s_ref[...] = jnp.dot(z2, wfc_ref[0:HH, 0], preferred_element_type=jnp.float32) + bfc_ref[...]
  t_ref[...] = jnp.dot(z2, wfc_ref[HH:2 * HH, 0], preferred_element_type=jnp.float32)


_tc_prep1 = pl.pallas_call(
    _tc_prep1_body,
    out_shape=[jax.ShapeDtypeStruct((NP, HH), jnp.float32),
               jax.ShapeDtypeStruct((NN, HH), jnp.float32)],
)

_tc_mid = pl.pallas_call(
    _tc_mid_body,
    out_shape=[jax.ShapeDtypeStruct((NP, HH), jnp.float32),
               jax.ShapeDtypeStruct((NN, HH), jnp.float32)],
)

_tc_fin = pl.pallas_call(
    _tc_fin_body,
    out_shape=[jax.ShapeDtypeStruct((NN,), jnp.float32),
               jax.ShapeDtypeStruct((NN,), jnp.float32)],
)


def kernel(x, edge_index, W1, b1, W2, b2, Wfc, bfc):
  src = edge_index[0]
  dst = edge_index[1]
  npad = EPAD - EE
  srcp = jnp.concatenate([src, jnp.zeros((npad,), jnp.int32)])
  dstp = jnp.concatenate([dst, jnp.full((npad,), NN, jnp.int32)])
  src3 = srcp.reshape(NWORK, NCH, CHUNK)
  dst3 = dstp.reshape(NWORK, NCH, CHUNK)
  ei3 = edge_index.reshape(2, NWORK, EW)

  zeros16 = jnp.zeros((NP, LL), jnp.float32)
  ones16 = jnp.ones((CHUNK, LL), jnp.float32)
  zeros64 = jnp.zeros((NP, HH), jnp.float32)

  cnt_part = _sc_degree(dst3, zeros16, ones16)
  g1, u1 = _tc_prep1(x, W1, b1, cnt_part)
  s1_part = _sc_msgpass(g1, src3, dst3, zeros64)
  g2, u2 = _tc_mid(s1_part, u1, W2, b2, cnt_part)
  s2_part = _sc_msgpass(g2, src3, dst3, zeros64)
  s_tab, t_tab = _tc_fin(s2_part, u2, Wfc, bfc, cnt_part)
  dec = _sc_decode(s_tab, t_tab, ei3)
  return dec.reshape(EE, 1)
